# baseline (device time: 39341 ns/iter reference)
import jax
import jax.numpy as jnp
from jax import lax
from jax.experimental import pallas as pl
from jax.experimental.pallas import tpu as pltpu

B, S, H_SHARD, D = 4, 512, 8, 64
K = H_SHARD * D
N = 1024
S_HALF = S // 2


def kernel(O, Wo):
    O2 = O.reshape(B * S, K)

    def body(o_ref, wo_ref, out_ref, acc_ref, send_ref, recv_ref,
             send_sem, recv_sem):
        my_x = lax.axis_index("x")
        my_y = lax.axis_index("y")
        my_z = lax.axis_index("z")
        partner = 1 - my_x

        barrier_sem = pltpu.get_barrier_semaphore()
        pl.semaphore_signal(
            barrier_sem, inc=1,
            device_id=(partner, my_y, my_z),
            device_id_type=pl.DeviceIdType.MESH,
        )
        pl.semaphore_wait(barrier_sem, 1)

        o = o_ref[...].astype(jnp.bfloat16)
        w = wo_ref[...].astype(jnp.bfloat16)
        acc_ref[...] = jnp.dot(o, w, preferred_element_type=jnp.float32)

        partner_s = partner * S_HALF
        for b in range(B):
            send_ref[b, :, :] = acc_ref[
                pl.ds(b * S + partner_s, S_HALF), :
            ].astype(jnp.bfloat16)

        rdma = pltpu.make_async_remote_copy(
            src_ref=send_ref,
            dst_ref=recv_ref,
            send_sem=send_sem,
            recv_sem=recv_sem,
            device_id=(partner, my_y, my_z),
            device_id_type=pl.DeviceIdType.MESH,
        )
        rdma.start()
        rdma.wait()

        my_s = my_x * S_HALF
        for b in range(B):
            out_ref[b, :, :] = (
                acc_ref[pl.ds(b * S + my_s, S_HALF), :]
                + recv_ref[b, :, :].astype(jnp.float32)
            )

    return pl.pallas_call(
        body,
        out_shape=jax.ShapeDtypeStruct((B, S_HALF, N), jnp.float32),
        in_specs=[
            pl.BlockSpec(memory_space=pltpu.VMEM),
            pl.BlockSpec(memory_space=pltpu.VMEM),
        ],
        out_specs=pl.BlockSpec(memory_space=pltpu.VMEM),
        scratch_shapes=[
            pltpu.VMEM((B * S, N), jnp.float32),
            pltpu.VMEM((B, S_HALF, N), jnp.bfloat16),
            pltpu.VMEM((B, S_HALF, N), jnp.bfloat16),
            pltpu.SemaphoreType.DMA,
            pltpu.SemaphoreType.DMA,
        ],
        compiler_params=pltpu.CompilerParams(collective_id=0),
    )(O2, Wo)


# device time: 36649 ns/iter; 1.0735x vs baseline; 1.0735x over previous
import jax
import jax.numpy as jnp
from jax import lax
from jax.experimental import pallas as pl
from jax.experimental.pallas import tpu as pltpu

B, S, H_SHARD, D = 4, 512, 8, 64
K = H_SHARD * D
N = 1024
S_HALF = S // 2


def kernel(O, Wo):
    O2 = O.reshape(B * S, K)

    def body(o_ref, wo_ref, out_ref, send_ref, recv_ref, send_sems, recv_sems):
        my_x = lax.axis_index("x")
        my_y = lax.axis_index("y")
        my_z = lax.axis_index("z")
        partner = 1 - my_x

        barrier_sem = pltpu.get_barrier_semaphore()
        pl.semaphore_signal(
            barrier_sem, inc=1,
            device_id=(partner, my_y, my_z),
            device_id_type=pl.DeviceIdType.MESH,
        )
        pl.semaphore_wait(barrier_sem, 1)

        w = wo_ref[...].astype(jnp.bfloat16)
        partner_s = partner * S_HALF
        my_s = my_x * S_HALF

        rdmas = []
        for b in range(B):
            ob = o_ref[pl.ds(b * S + partner_s, S_HALF), :].astype(jnp.bfloat16)
            send_ref[b, :, :] = jnp.dot(
                ob, w, preferred_element_type=jnp.float32
            ).astype(jnp.bfloat16)
            rdma = pltpu.make_async_remote_copy(
                src_ref=send_ref.at[b],
                dst_ref=recv_ref.at[b],
                send_sem=send_sems.at[b],
                recv_sem=recv_sems.at[b],
                device_id=(partner, my_y, my_z),
                device_id_type=pl.DeviceIdType.MESH,
            )
            rdma.start()
            rdmas.append(rdma)

        for b in range(B):
            ob = o_ref[pl.ds(b * S + my_s, S_HALF), :].astype(jnp.bfloat16)
            mine = jnp.dot(ob, w, preferred_element_type=jnp.float32)
            rdmas[b].wait_recv()
            out_ref[b, :, :] = mine + recv_ref[b, :, :].astype(jnp.float32)

        for b in range(B):
            rdmas[b].wait_send()

    return pl.pallas_call(
        body,
        out_shape=jax.ShapeDtypeStruct((B, S_HALF, N), jnp.float32),
        in_specs=[
            pl.BlockSpec(memory_space=pltpu.VMEM),
            pl.BlockSpec(memory_space=pltpu.VMEM),
        ],
        out_specs=pl.BlockSpec(memory_space=pltpu.VMEM),
        scratch_shapes=[
            pltpu.VMEM((B, S_HALF, N), jnp.bfloat16),
            pltpu.VMEM((B, S_HALF, N), jnp.bfloat16),
            pltpu.SemaphoreType.DMA((B,)),
            pltpu.SemaphoreType.DMA((B,)),
        ],
        compiler_params=pltpu.CompilerParams(collective_id=0),
    )(O2, Wo)


# device time: 35643 ns/iter; 1.1038x vs baseline; 1.0282x over previous
import jax
import jax.numpy as jnp
from jax import lax
from jax.experimental import pallas as pl
from jax.experimental.pallas import tpu as pltpu

B, S, H_SHARD, D = 4, 512, 8, 64
K = H_SHARD * D
N = 1024
S_HALF = S // 2


def kernel(O, Wo):
    def body(o_ref, wo_ref, out_ref, o_vmem, out_vmem, send_ref, recv_ref,
             load_sems, out_sems, send_sems, recv_sems):
        my_x = lax.axis_index("x")
        my_y = lax.axis_index("y")
        my_z = lax.axis_index("z")
        partner = 1 - my_x

        barrier_sem = pltpu.get_barrier_semaphore()
        pl.semaphore_signal(
            barrier_sem, inc=1,
            device_id=(partner, my_y, my_z),
            device_id_type=pl.DeviceIdType.MESH,
        )
        pl.semaphore_wait(barrier_sem, 1)

        partner_s = partner * S_HALF
        my_s = my_x * S_HALF

        loads = []
        for c in range(2 * B):
            b = c % B
            s0 = partner_s if c < B else my_s
            cp = pltpu.make_async_copy(
                o_ref.at[b, pl.ds(s0, S_HALF)], o_vmem.at[c], load_sems.at[c]
            )
            cp.start()
            loads.append(cp)

        w = wo_ref[...].astype(jnp.bfloat16)

        rdmas = []
        for b in range(B):
            loads[b].wait()
            ob = o_vmem[b].reshape(S_HALF, K).astype(jnp.bfloat16)
            send_ref[b, :, :] = jnp.dot(
                ob, w, preferred_element_type=jnp.float32
            ).astype(jnp.bfloat16)
            rdma = pltpu.make_async_remote_copy(
                src_ref=send_ref.at[b],
                dst_ref=recv_ref.at[b],
                send_sem=send_sems.at[b],
                recv_sem=recv_sems.at[b],
                device_id=(partner, my_y, my_z),
                device_id_type=pl.DeviceIdType.MESH,
            )
            rdma.start()
            rdmas.append(rdma)

        outs = []
        for b in range(B):
            loads[B + b].wait()
            ob = o_vmem[B + b].reshape(S_HALF, K).astype(jnp.bfloat16)
            mine = jnp.dot(ob, w, preferred_element_type=jnp.float32)
            rdmas[b].wait_recv()
            out_vmem[b, :, :] = mine + recv_ref[b, :, :].astype(jnp.float32)
            cp = pltpu.make_async_copy(
                out_vmem.at[b], out_ref.at[b], out_sems.at[b]
            )
            cp.start()
            outs.append(cp)

        for b in range(B):
            outs[b].wait()
            rdmas[b].wait_send()

    return pl.pallas_call(
        body,
        out_shape=jax.ShapeDtypeStruct((B, S_HALF, N), jnp.float32),
        in_specs=[
            pl.BlockSpec(memory_space=pltpu.MemorySpace.HBM),
            pl.BlockSpec(memory_space=pltpu.MemorySpace.VMEM),
        ],
        out_specs=pl.BlockSpec(memory_space=pltpu.MemorySpace.HBM),
        scratch_shapes=[
            pltpu.VMEM((2 * B, S_HALF, H_SHARD, D), jnp.float32),
            pltpu.VMEM((B, S_HALF, N), jnp.float32),
            pltpu.VMEM((B, S_HALF, N), jnp.bfloat16),
            pltpu.VMEM((B, S_HALF, N), jnp.bfloat16),
            pltpu.SemaphoreType.DMA((2 * B,)),
            pltpu.SemaphoreType.DMA((B,)),
            pltpu.SemaphoreType.DMA((B,)),
            pltpu.SemaphoreType.DMA((B,)),
        ],
        compiler_params=pltpu.CompilerParams(collective_id=0),
    )(O, Wo)


# device time: 34765 ns/iter; 1.1316x vs baseline; 1.0253x over previous
import jax
import jax.numpy as jnp
from jax import lax
from jax.experimental import pallas as pl
from jax.experimental.pallas import tpu as pltpu

B, S, H_SHARD, D = 4, 512, 8, 64
K = H_SHARD * D
N = 1024
S_HALF = S // 2


def kernel(O, Wo):
    O2 = O.reshape(B * S, H_SHARD, D)

    def body(o_ref, wo_ref, out_ref, send_ref, recv_ref, send_sems, recv_sems):
        my_x = lax.axis_index("x")
        my_y = lax.axis_index("y")
        my_z = lax.axis_index("z")
        partner = 1 - my_x

        barrier_sem = pltpu.get_barrier_semaphore()
        pl.semaphore_signal(
            barrier_sem, inc=1,
            device_id=(partner, my_y, my_z),
            device_id_type=pl.DeviceIdType.MESH,
        )
        pl.semaphore_wait(barrier_sem, 1)

        w = wo_ref[...].astype(jnp.bfloat16)
        partner_s = partner * S_HALF
        my_s = my_x * S_HALF

        rdmas = []
        for b in range(B):
            ob = (
                o_ref[pl.ds(b * S + partner_s, S_HALF), :, :]
                .reshape(S_HALF, K)
                .astype(jnp.bfloat16)
            )
            send_ref[b, :, :] = jnp.dot(
                ob, w, preferred_element_type=jnp.float32
            ).astype(jnp.bfloat16)
            rdma = pltpu.make_async_remote_copy(
                src_ref=send_ref.at[b],
                dst_ref=recv_ref.at[b],
                send_sem=send_sems.at[b],
                recv_sem=recv_sems.at[b],
                device_id=(partner, my_y, my_z),
                device_id_type=pl.DeviceIdType.MESH,
            )
            rdma.start()
            rdmas.append(rdma)

        for b in range(B):
            ob = (
                o_ref[pl.ds(b * S + my_s, S_HALF), :, :]
                .reshape(S_HALF, K)
                .astype(jnp.bfloat16)
            )
            mine = jnp.dot(ob, w, preferred_element_type=jnp.float32)
            rdmas[b].wait_recv()
            out_ref[b, :, :] = mine + recv_ref[b, :, :].astype(jnp.float32)

        for b in range(B):
            rdmas[b].wait_send()

    return pl.pallas_call(
        body,
        out_shape=jax.ShapeDtypeStruct((B, S_HALF, N), jnp.float32),
        in_specs=[
            pl.BlockSpec(memory_space=pltpu.MemorySpace.VMEM),
            pl.BlockSpec(memory_space=pltpu.MemorySpace.VMEM),
        ],
        out_specs=pl.BlockSpec(memory_space=pltpu.MemorySpace.VMEM),
        scratch_shapes=[
            pltpu.VMEM((B, S_HALF, N), jnp.bfloat16),
            pltpu.VMEM((B, S_HALF, N), jnp.bfloat16),
            pltpu.SemaphoreType.DMA((B,)),
            pltpu.SemaphoreType.DMA((B,)),
        ],
        compiler_params=pltpu.CompilerParams(collective_id=0),
    )(O2, Wo)
